# baseline (speedup 1.0000x reference)
import functools
import math

import jax
import jax.numpy as jnp
import numpy as np
from jax import lax
from jax.experimental import pallas as pl
from jax.experimental.pallas import tpu as pltpu
from jax.experimental.pallas import tpu_sc as plsc

BATCH = 4
MAX_SEQ = 8192
D_MODEL = 768
SCALE = 1.0 / math.sqrt(float(D_MODEL))

NC = 2
NS = 16
NW = NC * NS
S_PER_W = MAX_SEQ // NW
CHUNK = 32
N_CHUNKS = S_PER_W // CHUNK
LANES = 16
VECS_PER_ROW = D_MODEL // LANES


def _make_pos_encoding():
    position = np.arange(MAX_SEQ, dtype=np.float32).reshape(MAX_SEQ, 1)
    even_index = np.arange(0, D_MODEL, 2).astype(np.float32)
    denominator = np.power(10000.0, even_index / float(D_MODEL))
    even_pos = np.sin(position / denominator)
    odd_pos = np.cos(position / denominator)
    pe = np.stack([even_pos, odd_pos], axis=2).reshape(MAX_SEQ, D_MODEL)
    return jnp.asarray(pe, dtype=jnp.float32)


_MESH = plsc.VectorSubcoreMesh(core_axis_name="c", subcore_axis_name="s")


@functools.partial(
    pl.kernel,
    mesh=_MESH,
    out_type=jax.ShapeDtypeStruct((BATCH, MAX_SEQ, D_MODEL), jnp.float32),
    scratch_types=[
        pltpu.VMEM((BATCH, S_PER_W), jnp.int32),
        pltpu.VMEM((CHUNK, D_MODEL), jnp.float32),
        pltpu.VMEM((BATCH, CHUNK, D_MODEL), jnp.float32),
        pltpu.SemaphoreType.DMA,
        pltpu.SemaphoreType.DMA,
    ],
)
def _embed_kernel(x_hbm, table_hbm, pe_hbm, out_hbm,
                  idx_v, pe_v, g_v, gsem, ssem):
    wid = lax.axis_index("s") * NC + lax.axis_index("c")
    sbase = wid * S_PER_W

    for b in range(BATCH):
        pltpu.sync_copy(x_hbm.at[b, pl.ds(sbase, S_PER_W)], idx_v.at[b])

    def gather_start(c, b):
        pltpu.async_copy(
            table_hbm.at[idx_v.at[b, pl.ds(c * CHUNK, CHUNK)]],
            g_v.at[b], gsem)

    def gather_wait(c, b):
        pltpu.make_async_copy(
            table_hbm.at[idx_v.at[b, pl.ds(c * CHUNK, CHUNK)]],
            g_v.at[b], gsem).wait()

    def store_start(c, b):
        pltpu.async_copy(
            g_v.at[b], out_hbm.at[b, pl.ds(sbase + c * CHUNK, CHUNK)],
            ssem)

    def store_wait(c, b):
        pltpu.make_async_copy(
            g_v.at[b], out_hbm.at[b, pl.ds(sbase + c * CHUNK, CHUNK)],
            ssem).wait()

    def chunk_body(c, carry):
        @pl.when(c > 0)
        def _drain_prev_stores():
            for b in range(BATCH):
                store_wait(c - 1, b)

        for b in range(BATCH):
            gather_start(c, b)
        pltpu.sync_copy(pe_hbm.at[pl.ds(sbase + c * CHUNK, CHUNK)], pe_v)
        for b in range(BATCH):
            gather_wait(c, b)

        def row_body(r, rc):
            for j in range(VECS_PER_ROW):
                sl = pl.ds(j * LANES, LANES)
                pe_reg = pe_v[r, sl]
                for b in range(BATCH):
                    g_v[b, r, sl] = g_v[b, r, sl] * SCALE + pe_reg
            return rc

        lax.fori_loop(0, CHUNK, row_body, 0)
        for b in range(BATCH):
            store_start(c, b)
        return carry

    lax.fori_loop(0, N_CHUNKS, chunk_body, 0)

    for b in range(BATCH):
        store_wait(N_CHUNKS - 1, b)


def kernel(x, emb_table):
    pe = _make_pos_encoding()
    return _embed_kernel(x, emb_table, pe)


# speedup vs baseline: 1.6868x; 1.6868x over previous
import functools
import math

import jax
import jax.numpy as jnp
import numpy as np
from jax import lax
from jax.experimental import pallas as pl
from jax.experimental.pallas import tpu as pltpu
from jax.experimental.pallas import tpu_sc as plsc

BATCH = 4
MAX_SEQ = 8192
D_MODEL = 768
SCALE = 1.0 / math.sqrt(float(D_MODEL))

NC = 2
NS = 16
NW = NC * NS
S_PER_W = MAX_SEQ // NW
CHUNK = 32
N_CHUNKS = S_PER_W // CHUNK
LANES = 16
VECS_PER_ROW = D_MODEL // LANES


def _make_pos_encoding():
    position = np.arange(MAX_SEQ, dtype=np.float32).reshape(MAX_SEQ, 1)
    even_index = np.arange(0, D_MODEL, 2).astype(np.float32)
    denominator = np.power(10000.0, even_index / float(D_MODEL))
    even_pos = np.sin(position / denominator)
    odd_pos = np.cos(position / denominator)
    pe = np.stack([even_pos, odd_pos], axis=2).reshape(MAX_SEQ, D_MODEL)
    return jnp.asarray(pe, dtype=jnp.float32)


_MESH = plsc.VectorSubcoreMesh(core_axis_name="c", subcore_axis_name="s")


@functools.partial(
    pl.kernel,
    mesh=_MESH,
    out_type=jax.ShapeDtypeStruct((BATCH, MAX_SEQ, D_MODEL), jnp.float32),
    scratch_types=[
        pltpu.VMEM((BATCH, S_PER_W), jnp.int32),
        pltpu.VMEM((CHUNK, D_MODEL), jnp.float32),
        pltpu.VMEM((BATCH, CHUNK, D_MODEL), jnp.float32),
        pltpu.SemaphoreType.DMA,
        pltpu.SemaphoreType.DMA,
    ],
)
def _embed_kernel(x_hbm, table_hbm, pe_hbm, out_hbm,
                  idx_v, pe_v, g_v, gsem, ssem):
    wid = lax.axis_index("s") * NC + lax.axis_index("c")
    sbase = wid * S_PER_W

    for b in range(BATCH):
        pltpu.sync_copy(x_hbm.at[b, pl.ds(sbase, S_PER_W)], idx_v.at[b])

    def gather_start(c, b):
        pltpu.async_copy(
            table_hbm.at[idx_v.at[b, pl.ds(c * CHUNK, CHUNK)]],
            g_v.at[b], gsem)

    def gather_wait(c, b):
        pltpu.make_async_copy(
            table_hbm.at[idx_v.at[b, pl.ds(c * CHUNK, CHUNK)]],
            g_v.at[b], gsem).wait()

    def store_start(c, b):
        pltpu.async_copy(
            g_v.at[b], out_hbm.at[b, pl.ds(sbase + c * CHUNK, CHUNK)],
            ssem)

    def store_wait(c, b):
        pltpu.make_async_copy(
            g_v.at[b], out_hbm.at[b, pl.ds(sbase + c * CHUNK, CHUNK)],
            ssem).wait()

    def chunk_body(c, carry):
        @pl.when(c > 0)
        def _drain_prev_stores():
            for b in range(BATCH):
                store_wait(c - 1, b)

        for b in range(BATCH):
            gather_start(c, b)
        pltpu.sync_copy(pe_hbm.at[pl.ds(sbase + c * CHUNK, CHUNK)], pe_v)
        for b in range(BATCH):
            gather_wait(c, b)

        @plsc.parallel_loop(0, CHUNK)
        def row_body(r):
            for j0 in range(0, VECS_PER_ROW, 2):
                sls = [pl.ds((j0 + u) * LANES, LANES) for u in range(2)]
                pe_regs = [pe_v[r, sl] for sl in sls]
                g_regs = [[g_v[b, r, sl] for sl in sls] for b in range(BATCH)]
                res = [[g_regs[b][u] * SCALE + pe_regs[u] for u in range(2)]
                       for b in range(BATCH)]
                for b in range(BATCH):
                    for u in range(2):
                        g_v[b, r, sls[u]] = res[b][u]
        for b in range(BATCH):
            store_start(c, b)
        return carry

    lax.fori_loop(0, N_CHUNKS, chunk_body, 0)

    for b in range(BATCH):
        store_wait(N_CHUNKS - 1, b)


def kernel(x, emb_table):
    pe = _make_pos_encoding()
    return _embed_kernel(x, emb_table, pe)


# speedup vs baseline: 2.5816x; 1.5305x over previous
import functools
import math

import jax
import jax.numpy as jnp
import numpy as np
from jax import lax
from jax.experimental import pallas as pl
from jax.experimental.pallas import tpu as pltpu
from jax.experimental.pallas import tpu_sc as plsc

BATCH = 4
MAX_SEQ = 8192
D_MODEL = 768
SCALE = 1.0 / math.sqrt(float(D_MODEL))

NC = 2
NS = 16
NW = NC * NS
S_PER_W = MAX_SEQ // NW
CHUNK = 8
N_CHUNKS = S_PER_W // CHUNK
NSETS = 4
N_QUADS = N_CHUNKS // NSETS
LANES = 16
VECS_PER_ROW = D_MODEL // LANES


def _make_pos_encoding():
    position = np.arange(MAX_SEQ, dtype=np.float32).reshape(MAX_SEQ, 1)
    even_index = np.arange(0, D_MODEL, 2).astype(np.float32)
    denominator = np.power(10000.0, even_index / float(D_MODEL))
    even_pos = np.sin(position / denominator)
    odd_pos = np.cos(position / denominator)
    pe = np.stack([even_pos, odd_pos], axis=2).reshape(MAX_SEQ, D_MODEL)
    return jnp.asarray(pe, dtype=jnp.float32)


_MESH = plsc.VectorSubcoreMesh(core_axis_name="c", subcore_axis_name="s")


@functools.partial(
    pl.kernel,
    mesh=_MESH,
    out_type=jax.ShapeDtypeStruct((BATCH, MAX_SEQ, D_MODEL), jnp.float32),
    scratch_types=[
        pltpu.VMEM((BATCH, S_PER_W), jnp.int32),
        pltpu.VMEM((2, CHUNK, D_MODEL), jnp.float32),
        pltpu.VMEM((NSETS, BATCH, CHUNK, D_MODEL), jnp.float32),
    ] + [pltpu.SemaphoreType.DMA] * (NSETS + NSETS + 2),
)
def _embed_kernel(x_hbm, table_hbm, pe_hbm, out_hbm,
                  idx_v, pe_v, g_v, *sems):
    gsem = sems[:NSETS]
    ssem = sems[NSETS:2 * NSETS]
    psem = sems[2 * NSETS:]
    wid = lax.axis_index("s") * NC + lax.axis_index("c")
    sbase = wid * S_PER_W

    for b in range(BATCH):
        pltpu.sync_copy(x_hbm.at[b, pl.ds(sbase, S_PER_W)], idx_v.at[b])

    def gather_start(c, s):
        for b in range(BATCH):
            pltpu.async_copy(
                table_hbm.at[idx_v.at[b, pl.ds(c * CHUNK, CHUNK)]],
                g_v.at[s, b], gsem[s])

    def gather_wait(c, s):
        pltpu.make_async_copy(
            out_hbm.at[:, pl.ds(0, CHUNK)], g_v.at[s], gsem[s]).wait()

    def pe_start(c, s):
        pltpu.async_copy(
            pe_hbm.at[pl.ds(sbase + c * CHUNK, CHUNK)], pe_v.at[s % 2],
            psem[s % 2])

    def pe_wait(c, s):
        pltpu.make_async_copy(
            pe_hbm.at[pl.ds(sbase + c * CHUNK, CHUNK)], pe_v.at[s % 2],
            psem[s % 2]).wait()

    def store_start(c, s):
        pltpu.async_copy(
            g_v.at[s], out_hbm.at[:, pl.ds(sbase + c * CHUNK, CHUNK)],
            ssem[s])

    def store_wait(c, s):
        pltpu.make_async_copy(
            g_v.at[s], out_hbm.at[:, pl.ds(sbase + c * CHUNK, CHUNK)],
            ssem[s]).wait()

    def compute(s):
        def row_body(r, carry):
            for j0 in range(0, VECS_PER_ROW, 2):
                sls = [pl.ds((j0 + u) * LANES, LANES) for u in range(2)]
                pe_regs = [pe_v[s % 2, r, sl] for sl in sls]
                g_regs = [[g_v[s, b, r, sl] for sl in sls]
                          for b in range(BATCH)]
                res = [[g_regs[b][u] * SCALE + pe_regs[u] for u in range(2)]
                       for b in range(BATCH)]
                for b in range(BATCH):
                    for u in range(2):
                        g_v[s, b, r, sls[u]] = res[b][u]
            return carry

        lax.fori_loop(0, CHUNK, row_body, 0)

    gather_start(0, 0)
    pe_start(0, 0)

    def quad_body(q, carry):
        for cc in range(NSETS):
            c = q * NSETS + cc
            ns = (cc + 1) % NSETS

            gather_wait(c, cc)
            pe_wait(c, cc)

            if cc == NSETS - 1:
                store_wait(c - (NSETS - 1), ns)

                @pl.when(q < N_QUADS - 1)
                def _fire_next_last():
                    gather_start(c + 1, ns)
                    pe_start(c + 1, ns)
            else:
                @pl.when(q > 0)
                def _drain_next_set():
                    store_wait(c - (NSETS - 1), ns)

                gather_start(c + 1, ns)
                pe_start(c + 1, ns)

            compute(cc)
            store_start(c, cc)
        return carry

    lax.fori_loop(0, N_QUADS, quad_body, 0)

    for cc in range(1, NSETS):
        store_wait(N_CHUNKS - NSETS + cc, cc)


def kernel(x, emb_table):
    pe = _make_pos_encoding()
    return _embed_kernel(x, emb_table, pe)
